# baseline (speedup 1.0000x reference)
import functools
import math

import jax
import jax.numpy as jnp
from jax import lax
from jax.experimental import pallas as pl
from jax.experimental.pallas import tpu as pltpu
from jax.experimental.pallas import tpu_sc as plsc


_NC = 2
_NS = 16
_NW = _NC * _NS

_CH = 128


def _scale_table(table, scale):
    v, d = table.shape
    block = 800
    assert v % block == 0

    def body(t_ref, o_ref):
        o_ref[...] = t_ref[...] * scale

    return pl.pallas_call(
        body,
        grid=(v // block,),
        in_specs=[pl.BlockSpec((block, d), lambda i: (i, 0))],
        out_specs=pl.BlockSpec((block, d), lambda i: (i, 0)),
        out_shape=jax.ShapeDtypeStruct((v, d), table.dtype),
    )(table)


_NB = 2
_GPS = 2


def _make_sc_gather(n, v, d):
    sch = _CH * _GPS
    assert n % (_NW * sch * _NB) == 0
    chunks_per_w = n // (_NW * _CH)
    sc_per_w = chunks_per_w // _GPS
    mesh = plsc.VectorSubcoreMesh(core_axis_name="c", subcore_axis_name="s")

    @functools.partial(
        pl.kernel,
        mesh=mesh,
        out_type=jax.ShapeDtypeStruct((n, d), jnp.float32),
        scratch_types=[
            pltpu.VMEM((chunks_per_w, _CH), jnp.int32),
            [pltpu.VMEM((sch, d), jnp.float32)] * _NB,
            [[pltpu.SemaphoreType.DMA] * _GPS] * _NB,
            [pltpu.SemaphoreType.DMA] * _NB,
        ],
    )
    def k(table_hbm, idx_hbm, out_hbm, idx_v, rows, gsems, ssems):
        wid = lax.axis_index("s") * _NC + lax.axis_index("c")
        base = wid * (chunks_per_w * _CH)
        pltpu.sync_copy(idx_hbm.at[wid], idx_v)

        def fill(sb, sc):
            for h in range(_GPS):
                pltpu.async_copy(
                    table_hbm.at[idx_v.at[sc * _GPS + h]],
                    rows[sb].at[pl.ds(h * _CH, _CH)],
                    gsems[sb][h],
                )

        for sb in range(_NB):
            fill(sb, sb)

        def body(s, _):
            for sb in range(_NB):
                for h in range(_GPS):
                    pltpu.make_async_copy(
                        table_hbm.at[idx_v.at[0]],
                        rows[sb].at[pl.ds(h * _CH, _CH)],
                        gsems[sb][h],
                    ).wait()
                pltpu.async_copy(
                    rows[sb],
                    out_hbm.at[pl.ds(base + (s + sb) * sch, sch)],
                    ssems[sb],
                )
            for sb in range(_NB):
                s_next = s + _NB + sb

                @pl.when(s_next < sc_per_w)
                def _():
                    pltpu.make_async_copy(
                        rows[sb], out_hbm.at[pl.ds(base, sch)], ssems[sb]
                    ).wait()
                    fill(sb, s_next)

            return 0

        lax.fori_loop(0, sc_per_w // _NB, lambda s, c: body(s * _NB, c), 0)

        for sb in range(_NB):
            pltpu.make_async_copy(
                rows[sb], out_hbm.at[pl.ds(base, sch)], ssems[sb]
            ).wait()

    return k


def kernel(tokens, table):
    b, l = tokens.shape
    v, d = table.shape
    n = b * l
    scaled = _scale_table(table, math.sqrt(d))
    idx = tokens.reshape(_NW, n // (_NW * _CH), _CH).astype(jnp.int32)
    out = _make_sc_gather(n, v, d)(scaled, idx)
    return out.reshape(b, l, d)


# speedup vs baseline: 1.2596x; 1.2596x over previous
import functools
import math

import jax
import jax.numpy as jnp
from jax import lax
from jax.experimental import pallas as pl
from jax.experimental.pallas import tpu as pltpu
from jax.experimental.pallas import tpu_sc as plsc


_NC = 2
_NS = 16
_NW = _NC * _NS

_CH = 128


_NB = 2
_GPS = 2


def _make_sc_gather(n, v, d):
    scale = jnp.float32(math.sqrt(d))
    sch = _CH * _GPS
    assert n % (_NW * sch * _NB) == 0 and d % 16 == 0
    chunks_per_w = n // (_NW * _CH)
    sc_per_w = chunks_per_w // _GPS
    mesh = plsc.VectorSubcoreMesh(core_axis_name="c", subcore_axis_name="s")

    @functools.partial(
        pl.kernel,
        mesh=mesh,
        out_type=jax.ShapeDtypeStruct((n, d), jnp.float32),
        scratch_types=[
            pltpu.VMEM((chunks_per_w, _CH), jnp.int32),
            [pltpu.VMEM((sch, d), jnp.float32)] * _NB,
            [[pltpu.SemaphoreType.DMA] * _GPS] * _NB,
            [pltpu.SemaphoreType.DMA] * _NB,
        ],
    )
    def k(table_hbm, idx_hbm, out_hbm, idx_v, rows, gsems, ssems):
        wid = lax.axis_index("s") * _NC + lax.axis_index("c")
        base = wid * (chunks_per_w * _CH)
        pltpu.sync_copy(idx_hbm.at[wid], idx_v)

        def fill(sb, sc):
            for h in range(_GPS):
                pltpu.async_copy(
                    table_hbm.at[idx_v.at[sc * _GPS + h]],
                    rows[sb].at[pl.ds(h * _CH, _CH)],
                    gsems[sb][h],
                )

        for sb in range(_NB):
            fill(sb, sb)

        def body(s, _):
            for sb in range(_NB):
                for h in range(_GPS):
                    pltpu.make_async_copy(
                        table_hbm.at[idx_v.at[0]],
                        rows[sb].at[pl.ds(h * _CH, _CH)],
                        gsems[sb][h],
                    ).wait()

                def scale_row(r, _, sb=sb):
                    for j in range(d // 16):
                        sl = (r, pl.ds(j * 16, 16))
                        rows[sb][sl] = rows[sb][sl] * scale
                    return 0

                lax.fori_loop(0, sch, scale_row, 0)
                pltpu.async_copy(
                    rows[sb],
                    out_hbm.at[pl.ds(base + (s + sb) * sch, sch)],
                    ssems[sb],
                )
            for sb in range(_NB):
                s_next = s + _NB + sb

                @pl.when(s_next < sc_per_w)
                def _():
                    pltpu.make_async_copy(
                        rows[sb], out_hbm.at[pl.ds(base, sch)], ssems[sb]
                    ).wait()
                    fill(sb, s_next)

            return 0

        lax.fori_loop(0, sc_per_w // _NB, lambda s, c: body(s * _NB, c), 0)

        for sb in range(_NB):
            pltpu.make_async_copy(
                rows[sb], out_hbm.at[pl.ds(base, sch)], ssems[sb]
            ).wait()

    return k


def kernel(tokens, table):
    b, l = tokens.shape
    v, d = table.shape
    n = b * l
    idx = tokens.reshape(_NW, n // (_NW * _CH), _CH).astype(jnp.int32)
    out = _make_sc_gather(n, v, d)(table, idx)
    return out.reshape(b, l, d)
